# baseline (speedup 1.0000x reference)
import jax
import jax.numpy as jnp
from jax.experimental import pallas as pl
from jax.experimental.pallas import tpu as pltpu

_TOP_K = 2
_E = 8
_F = 768
_R = 16
_ER = _E * _R

_DN_RHS_T = (((1,), (1,)), ((), ()))


def _body(a_ref, b_ref, p_ref, tok_ref, out_ref, bs_s):
    @pl.when(pl.program_id(0) == 0)
    def _prep():
        b3 = b_ref[...].reshape(_E, _F, _R)
        bs_s[...] = jnp.transpose(b3, (0, 2, 1)).reshape(_ER, _F)

    tok = tok_ref[...]
    prows = jnp.concatenate(
        [jnp.broadcast_to(p_ref[e : e + 1, :], (_R, _F)) for e in range(_E)],
        axis=0,
    )
    simw = jnp.abs(jax.lax.dot_general(tok, prows, _DN_RHS_T,
                                       preferred_element_type=jnp.float32))
    m1 = jnp.max(simw, axis=1, keepdims=True)
    masked = jnp.where(simw == m1, -jnp.inf, simw)
    m2 = jnp.max(masked, axis=1, keepdims=True)
    e2 = jnp.exp(m2 - m1)
    denom = 1.0 + e2
    cexp = jnp.where(simw >= m2, jnp.exp(simw - m1), 0.0) / denom
    u = jax.lax.dot_general(tok, a_ref[...], _DN_RHS_T,
                            preferred_element_type=jnp.float32)
    v = u * cexp
    out_ref[...] = jnp.dot(v, bs_s[...], preferred_element_type=jnp.float32)


def kernel(x, lora_A, lora_B, prototypes, scaling):
    orig_shape = x.shape
    f_in = x.shape[-1]
    tok = x.reshape(-1, f_in)
    t = tok.shape[0]
    a2d = lora_A.reshape(_ER, _F)
    scalf = jnp.asarray(scaling, jnp.float32)
    b2 = (lora_B * scalf).reshape(_E, _F * _R)

    bt = 1024 if t % 1024 == 0 else t
    grid = (t // bt,)
    delta = pl.pallas_call(
        _body,
        grid=grid,
        in_specs=[
            pl.BlockSpec((_ER, _F), lambda i: (0, 0)),
            pl.BlockSpec((_E, _F * _R), lambda i: (0, 0)),
            pl.BlockSpec((_E, _F), lambda i: (0, 0)),
            pl.BlockSpec((bt, _F), lambda i: (i, 0)),
        ],
        out_specs=pl.BlockSpec((bt, _F), lambda i: (i, 0)),
        out_shape=jax.ShapeDtypeStruct((t, _F), jnp.float32),
        scratch_shapes=[pltpu.VMEM((_ER, _F), jnp.float32)],
    )(a2d, b2, prototypes, tok)
    return delta.reshape(orig_shape[:-1] + (_F,))


# speedup vs baseline: 1.5257x; 1.5257x over previous
import jax
import jax.numpy as jnp
from jax.experimental import pallas as pl

_TOP_K = 2
_E = 8
_F = 768
_R = 16
_ER = _E * _R

_DN_RHS_T = (((1,), (1,)), ((), ()))


def _body(a_ref, bs_ref, p_ref, tok_ref, out_ref):
    tok = tok_ref[...]
    prows = jnp.concatenate(
        [jnp.broadcast_to(p_ref[e : e + 1, :], (_R, _F)) for e in range(_E)],
        axis=0,
    )
    simw = jnp.abs(jax.lax.dot_general(tok, prows, _DN_RHS_T,
                                       preferred_element_type=jnp.float32))
    m1 = jnp.max(simw, axis=1, keepdims=True)
    masked = jnp.where(simw == m1, -jnp.inf, simw)
    m2 = jnp.max(masked, axis=1, keepdims=True)
    e2 = jnp.exp(m2 - m1)
    denom = 1.0 + e2
    cexp = jnp.where(simw >= m2, jnp.exp(simw - m1), 0.0) / denom
    u = jax.lax.dot_general(tok, a_ref[...], _DN_RHS_T,
                            preferred_element_type=jnp.float32)
    v = u * cexp
    out_ref[...] = jnp.dot(v, bs_ref[...], preferred_element_type=jnp.float32)


def kernel(x, lora_A, lora_B, prototypes, scaling):
    orig_shape = x.shape
    f_in = x.shape[-1]
    tok = x.reshape(-1, f_in)
    t = tok.shape[0]
    a2d = lora_A.reshape(_ER, _F)
    scalf = jnp.asarray(scaling, jnp.float32)
    bs = (lora_B * scalf).transpose(0, 2, 1).reshape(_ER, _F)

    bt = 1024 if t % 1024 == 0 else t
    grid = (t // bt,)
    delta = pl.pallas_call(
        _body,
        grid=grid,
        in_specs=[
            pl.BlockSpec((_ER, _F), lambda i: (0, 0)),
            pl.BlockSpec((_ER, _F), lambda i: (0, 0)),
            pl.BlockSpec((_E, _F), lambda i: (0, 0)),
            pl.BlockSpec((bt, _F), lambda i: (i, 0)),
        ],
        out_specs=pl.BlockSpec((bt, _F), lambda i: (i, 0)),
        out_shape=jax.ShapeDtypeStruct((t, _F), jnp.float32),
    )(a2d, bs, prototypes, tok)
    return delta.reshape(orig_shape[:-1] + (_F,))
